# baseline (speedup 1.0000x reference)
import functools

import jax
import jax.numpy as jnp
from jax import lax
from jax.experimental import pallas as pl
from jax.experimental.pallas import tpu as pltpu
from jax.experimental.pallas import tpu_sc as plsc

NUM_EXPERTS = 64
K_TOP = 8
HIDDEN = 4096
TOKENS = 16384

_BT = 1024

_NC = 2
_NS = 16
_L = 16
_NW = _NC * _NS
_CHUNK = TOKENS // _NW
_G = _CHUNK // _L


def _scores_body(x_ref, w_ref, o_ref):
    logits = lax.dot_general(
        x_ref[...], w_ref[...],
        (((1,), (1,)), ((), ())),
        preferred_element_type=jnp.float32,
    )
    m = jnp.max(logits, axis=1, keepdims=True)
    p = jnp.exp(logits - m)
    o_ref[...] = p / jnp.sum(p, axis=1, keepdims=True)


@functools.partial(
    pl.kernel,
    out_type=(
        jax.ShapeDtypeStruct((TOKENS, K_TOP), jnp.int32),
        jax.ShapeDtypeStruct((TOKENS, K_TOP), jnp.float32),
    ),
    mesh=plsc.VectorSubcoreMesh(core_axis_name="c", subcore_axis_name="s"),
    compiler_params=pltpu.CompilerParams(
        needs_layout_passes=False, use_tc_tiling_on_sc=False),
    scratch_types=[
        pltpu.VMEM((_CHUNK, NUM_EXPERTS), jnp.float32),
        pltpu.VMEM((_CHUNK, K_TOP), jnp.int32),
        pltpu.VMEM((_CHUNK, K_TOP), jnp.float32),
    ],
)
def _sc_topk(scores_hbm, idx_hbm, w_hbm, sc_v, idx_v, w_v):
    wid = lax.axis_index("s") * _NC + lax.axis_index("c")
    base = wid * _CHUNK
    pltpu.sync_copy(scores_hbm.at[pl.ds(base, _CHUNK)], sc_v)

    def group(g, carry):
        tok = lax.iota(jnp.int32, _L) + g * _L
        val = [jnp.full((_L,), -1.0, jnp.float32) for _ in range(K_TOP)]
        idx = [jnp.zeros((_L,), jnp.int32) for _ in range(K_TOP)]
        for e in range(NUM_EXPERTS):
            ev = jnp.full((_L,), e, jnp.int32)
            v = plsc.load_gather(sc_v, [tok, ev])
            c = [v > val[i] for i in range(K_TOP)]
            nval, nidx = [], []
            for i in range(K_TOP):
                if i == 0:
                    nval.append(jnp.where(c[0], v, val[0]))
                    nidx.append(jnp.where(c[0], ev, idx[0]))
                else:
                    sv = jnp.where(c[i - 1], val[i - 1], v)
                    si = jnp.where(c[i - 1], idx[i - 1], ev)
                    nval.append(jnp.where(c[i], sv, val[i]))
                    nidx.append(jnp.where(c[i], si, idx[i]))
            val, idx = nval, nidx
        for j in range(K_TOP):
            jv = jnp.full((_L,), j, jnp.int32)
            plsc.store_scatter(idx_v, [tok, jv], idx[j])
            plsc.store_scatter(w_v, [tok, jv], val[j])
        return carry

    lax.fori_loop(0, _G, group, 0)
    pltpu.sync_copy(idx_v, idx_hbm.at[pl.ds(base, _CHUNK)])
    pltpu.sync_copy(w_v, w_hbm.at[pl.ds(base, _CHUNK)])


def kernel(hidden_states, weight):
    scores = pl.pallas_call(
        _scores_body,
        grid=(TOKENS // _BT,),
        in_specs=[
            pl.BlockSpec((_BT, HIDDEN), lambda i: (i, 0)),
            pl.BlockSpec((NUM_EXPERTS, HIDDEN), lambda i: (0, 0)),
        ],
        out_specs=pl.BlockSpec((_BT, NUM_EXPERTS), lambda i: (i, 0)),
        out_shape=jax.ShapeDtypeStruct((TOKENS, NUM_EXPERTS), jnp.float32),
    )(hidden_states, weight)
    return _sc_topk(scores)


# speedup vs baseline: 1.0876x; 1.0876x over previous
import functools

import jax
import jax.numpy as jnp
from jax import lax
from jax.experimental import pallas as pl
from jax.experimental.pallas import tpu as pltpu
from jax.experimental.pallas import tpu_sc as plsc

NUM_EXPERTS = 64
K_TOP = 8
HIDDEN = 4096
TOKENS = 16384

_BT = 1024

_NC = 2
_NS = 16
_L = 16
_NW = _NC * _NS
_CHUNK = TOKENS // _NW
_G = _CHUNK // _L


def _scores_body(x_ref, w_ref, o_ref):
    logits = lax.dot_general(
        x_ref[...], w_ref[...],
        (((1,), (1,)), ((), ())),
        preferred_element_type=jnp.float32,
    )
    m = jnp.max(logits, axis=1, keepdims=True)
    p = jnp.exp(logits - m)
    o_ref[...] = p / jnp.sum(p, axis=1, keepdims=True)


@functools.partial(
    pl.kernel,
    out_type=(
        jax.ShapeDtypeStruct((TOKENS, K_TOP), jnp.int32),
        jax.ShapeDtypeStruct((TOKENS, K_TOP), jnp.float32),
    ),
    mesh=plsc.VectorSubcoreMesh(core_axis_name="c", subcore_axis_name="s"),
    compiler_params=pltpu.CompilerParams(
        needs_layout_passes=False, use_tc_tiling_on_sc=False),
    scratch_types=[
        pltpu.VMEM((_CHUNK, NUM_EXPERTS), jnp.float32),
        pltpu.VMEM((_CHUNK, K_TOP), jnp.int32),
        pltpu.VMEM((_CHUNK, K_TOP), jnp.float32),
    ],
)
def _sc_topk(scores_hbm, idx_hbm, w_hbm, sc_v, idx_v, w_v):
    wid = lax.axis_index("s") * _NC + lax.axis_index("c")
    base = wid * _CHUNK
    pltpu.sync_copy(scores_hbm.at[pl.ds(base, _CHUNK)], sc_v)

    def group(g, carry):
        tok = lax.iota(jnp.int32, _L) + g * _L
        keys = []
        for e in range(NUM_EXPERTS):
            ev = jnp.full((_L,), e, jnp.int32)
            v = plsc.load_gather(sc_v, [tok, ev])
            b = plsc.bitcast(v, jnp.uint32)
            keys.append((b & jnp.uint32(0xFFFFFFC0)) | jnp.uint32(63 - e))

        def ce(a, i, j):
            hi = jnp.maximum(a[i], a[j])
            lo = jnp.minimum(a[i], a[j])
            a[i], a[j] = hi, lo

        s8 = ((0, 1), (2, 3), (4, 5), (6, 7),
              (0, 2), (1, 3), (4, 6), (5, 7),
              (1, 2), (5, 6), (0, 4), (3, 7),
              (1, 5), (2, 6), (1, 4), (3, 6),
              (2, 4), (3, 5), (3, 4))
        blocks = []
        for blk in range(NUM_EXPERTS // 8):
            a = keys[8 * blk:8 * blk + 8]
            for i, j in s8:
                ce(a, i, j)
            blocks.append(a)
        bit12 = ((0, 4), (1, 5), (2, 6), (3, 7),
                 (0, 2), (1, 3), (4, 6), (5, 7),
                 (0, 1), (2, 3), (4, 5), (6, 7))
        while len(blocks) > 1:
            nxt = []
            for p in range(0, len(blocks), 2):
                a, b = blocks[p], blocks[p + 1]
                c = [jnp.maximum(a[i], b[7 - i]) for i in range(8)]
                for i, j in bit12:
                    ce(c, i, j)
                nxt.append(c)
            blocks = nxt
        top = blocks[0]
        for j in range(K_TOP):
            jv = jnp.full((_L,), j, jnp.int32)
            ij = jnp.int32(63) - plsc.bitcast(top[j] & jnp.uint32(63), jnp.int32)
            wj = plsc.load_gather(sc_v, [tok, ij])
            plsc.store_scatter(idx_v, [tok, jv], ij)
            plsc.store_scatter(w_v, [tok, jv], wj)
        return carry

    lax.fori_loop(0, _G, group, 0)
    pltpu.sync_copy(idx_v, idx_hbm.at[pl.ds(base, _CHUNK)])
    pltpu.sync_copy(w_v, w_hbm.at[pl.ds(base, _CHUNK)])


def kernel(hidden_states, weight):
    scores = pl.pallas_call(
        _scores_body,
        grid=(TOKENS // _BT,),
        in_specs=[
            pl.BlockSpec((_BT, HIDDEN), lambda i: (i, 0)),
            pl.BlockSpec((NUM_EXPERTS, HIDDEN), lambda i: (0, 0)),
        ],
        out_specs=pl.BlockSpec((_BT, NUM_EXPERTS), lambda i: (i, 0)),
        out_shape=jax.ShapeDtypeStruct((TOKENS, NUM_EXPERTS), jnp.float32),
    )(hidden_states, weight)
    return _sc_topk(scores)


# speedup vs baseline: 1.1402x; 1.0484x over previous
import functools

import jax
import jax.numpy as jnp
from jax import lax
from jax.experimental import pallas as pl
from jax.experimental.pallas import tpu as pltpu
from jax.experimental.pallas import tpu_sc as plsc

NUM_EXPERTS = 64
K_TOP = 8
HIDDEN = 4096
TOKENS = 16384

_BT = 1024
_C = 4
_CT = TOKENS // _C

_NC = 2
_NS = 16
_L = 16
_NW = _NC * _NS


def _scores_body(x_ref, w_ref, o_ref):
    logits = lax.dot_general(
        x_ref[...], w_ref[...],
        (((1,), (1,)), ((), ())),
        preferred_element_type=jnp.float32,
    )
    m = jnp.max(logits, axis=1, keepdims=True)
    p = jnp.exp(logits - m)
    o_ref[...] = p / jnp.sum(p, axis=1, keepdims=True)


def _topk_group(sc_v, idx_v, w_v, g):
    tok = lax.iota(jnp.int32, _L) + g * _L
    keys = []
    for e in range(NUM_EXPERTS):
        ev = jnp.full((_L,), e, jnp.int32)
        v = plsc.load_gather(sc_v, [tok, ev])
        b = plsc.bitcast(v, jnp.uint32)
        keys.append((b & jnp.uint32(0xFFFFFFC0)) | jnp.uint32(63 - e))

    def ce(a, i, j):
        hi = jnp.maximum(a[i], a[j])
        lo = jnp.minimum(a[i], a[j])
        a[i], a[j] = hi, lo

    s8 = ((0, 1), (2, 3), (4, 5), (6, 7),
          (0, 2), (1, 3), (4, 6), (5, 7),
          (1, 2), (5, 6), (0, 4), (3, 7),
          (1, 5), (2, 6), (1, 4), (3, 6),
          (2, 4), (3, 5), (3, 4))
    blocks = []
    for blk in range(NUM_EXPERTS // 8):
        a = keys[8 * blk:8 * blk + 8]
        for i, j in s8:
            ce(a, i, j)
        blocks.append(a)
    bit12 = ((0, 4), (1, 5), (2, 6), (3, 7),
             (0, 2), (1, 3), (4, 6), (5, 7),
             (0, 1), (2, 3), (4, 5), (6, 7))
    while len(blocks) > 1:
        nxt = []
        for p in range(0, len(blocks), 2):
            a, b = blocks[p], blocks[p + 1]
            c = [jnp.maximum(a[i], b[7 - i]) for i in range(8)]
            for i, j in bit12:
                ce(c, i, j)
            nxt.append(c)
        blocks = nxt
    top = blocks[0]
    for j in range(K_TOP):
        jv = jnp.full((_L,), j, jnp.int32)
        ij = jnp.int32(63) - plsc.bitcast(top[j] & jnp.uint32(63), jnp.int32)
        wj = plsc.load_gather(sc_v, [tok, ij])
        plsc.store_scatter(idx_v, [tok, jv], ij)
        plsc.store_scatter(w_v, [tok, jv], wj)


def _make_sc_topk(ntok):
    chunk = ntok // _NW
    ngroups = chunk // _L

    @functools.partial(
        pl.kernel,
        out_type=(
            jax.ShapeDtypeStruct((ntok, K_TOP), jnp.int32),
            jax.ShapeDtypeStruct((ntok, K_TOP), jnp.float32),
        ),
        mesh=plsc.VectorSubcoreMesh(core_axis_name="c", subcore_axis_name="s"),
        compiler_params=pltpu.CompilerParams(
            needs_layout_passes=False, use_tc_tiling_on_sc=False),
        scratch_types=[
            pltpu.VMEM((chunk, NUM_EXPERTS), jnp.float32),
            pltpu.VMEM((chunk, K_TOP), jnp.int32),
            pltpu.VMEM((chunk, K_TOP), jnp.float32),
        ],
    )
    def sc_topk(scores_hbm, idx_hbm, w_hbm, sc_v, idx_v, w_v):
        wid = lax.axis_index("s") * _NC + lax.axis_index("c")
        base = wid * chunk
        pltpu.sync_copy(scores_hbm.at[pl.ds(base, chunk)], sc_v)

        def group(g, carry):
            _topk_group(sc_v, idx_v, w_v, g)
            return carry

        lax.fori_loop(0, ngroups, group, 0)
        pltpu.sync_copy(idx_v, idx_hbm.at[pl.ds(base, chunk)])
        pltpu.sync_copy(w_v, w_hbm.at[pl.ds(base, chunk)])

    return sc_topk


_sc_topk_chain = _make_sc_topk(_CT)


def _make_scores_call(c):
    nsteps = _CT // _BT

    return pl.pallas_call(
        _scores_body,
        grid=(nsteps,),
        in_specs=[
            pl.BlockSpec((_BT, HIDDEN), lambda i, c=c: (c * nsteps + i, 0)),
            pl.BlockSpec((NUM_EXPERTS, HIDDEN), lambda i: (0, 0)),
        ],
        out_specs=pl.BlockSpec((_BT, NUM_EXPERTS), lambda i: (i, 0)),
        out_shape=jax.ShapeDtypeStruct((_CT, NUM_EXPERTS), jnp.float32),
    )


def kernel(hidden_states, weight):
    idxs, ws = [], []
    for c in range(_C):
        scores_c = _make_scores_call(c)(hidden_states, weight)
        i_c, w_c = _sc_topk_chain(scores_c)
        idxs.append(i_c)
        ws.append(w_c)
    return jnp.concatenate(idxs, 0), jnp.concatenate(ws, 0)


# speedup vs baseline: 1.1471x; 1.0060x over previous
import functools

import jax
import jax.numpy as jnp
from jax import lax
from jax.experimental import pallas as pl
from jax.experimental.pallas import tpu as pltpu
from jax.experimental.pallas import tpu_sc as plsc

NUM_EXPERTS = 64
K_TOP = 8
HIDDEN = 4096
TOKENS = 16384

_BT = 1024
_CHAIN = (5120, 5120, 5120, 1024)

_NC = 2
_NS = 16
_L = 16
_NW = _NC * _NS


def _scores_body(x_ref, w_ref, o_ref):
    logits = lax.dot_general(
        x_ref[...], w_ref[...],
        (((1,), (1,)), ((), ())),
        preferred_element_type=jnp.float32,
    )
    m = jnp.max(logits, axis=1, keepdims=True)
    p = jnp.exp(logits - m)
    o_ref[...] = p / jnp.sum(p, axis=1, keepdims=True)


def _topk_group(sc_v, idx_v, w_v, g):
    tok = lax.iota(jnp.int32, _L) + g * _L
    keys = []
    for e in range(NUM_EXPERTS):
        ev = jnp.full((_L,), e, jnp.int32)
        v = plsc.load_gather(sc_v, [tok, ev])
        b = plsc.bitcast(v, jnp.uint32)
        keys.append((b & jnp.uint32(0xFFFFFFC0)) | jnp.uint32(63 - e))

    def ce(a, i, j):
        hi = jnp.maximum(a[i], a[j])
        lo = jnp.minimum(a[i], a[j])
        a[i], a[j] = hi, lo

    s8 = ((0, 1), (2, 3), (4, 5), (6, 7),
          (0, 2), (1, 3), (4, 6), (5, 7),
          (1, 2), (5, 6), (0, 4), (3, 7),
          (1, 5), (2, 6), (1, 4), (3, 6),
          (2, 4), (3, 5), (3, 4))
    blocks = []
    for blk in range(NUM_EXPERTS // 8):
        a = keys[8 * blk:8 * blk + 8]
        for i, j in s8:
            ce(a, i, j)
        blocks.append(a)
    bit12 = ((0, 4), (1, 5), (2, 6), (3, 7),
             (0, 2), (1, 3), (4, 6), (5, 7),
             (0, 1), (2, 3), (4, 5), (6, 7))
    while len(blocks) > 1:
        nxt = []
        for p in range(0, len(blocks), 2):
            a, b = blocks[p], blocks[p + 1]
            c = [jnp.maximum(a[i], b[7 - i]) for i in range(8)]
            for i, j in bit12:
                ce(c, i, j)
            nxt.append(c)
        blocks = nxt
    top = blocks[0]
    for j in range(K_TOP):
        jv = jnp.full((_L,), j, jnp.int32)
        ij = jnp.int32(63) - plsc.bitcast(top[j] & jnp.uint32(63), jnp.int32)
        wj = plsc.load_gather(sc_v, [tok, ij])
        plsc.store_scatter(idx_v, [tok, jv], ij)
        plsc.store_scatter(w_v, [tok, jv], wj)


def _make_sc_topk(ntok):
    chunk = ntok // _NW
    ngroups = chunk // _L

    @functools.partial(
        pl.kernel,
        out_type=(
            jax.ShapeDtypeStruct((ntok, K_TOP), jnp.int32),
            jax.ShapeDtypeStruct((ntok, K_TOP), jnp.float32),
        ),
        mesh=plsc.VectorSubcoreMesh(core_axis_name="c", subcore_axis_name="s"),
        compiler_params=pltpu.CompilerParams(
            needs_layout_passes=False, use_tc_tiling_on_sc=False),
        scratch_types=[
            pltpu.VMEM((chunk, NUM_EXPERTS), jnp.float32),
            pltpu.VMEM((chunk, K_TOP), jnp.int32),
            pltpu.VMEM((chunk, K_TOP), jnp.float32),
        ],
    )
    def sc_topk(scores_hbm, idx_hbm, w_hbm, sc_v, idx_v, w_v):
        wid = lax.axis_index("s") * _NC + lax.axis_index("c")
        base = wid * chunk
        pltpu.sync_copy(scores_hbm.at[pl.ds(base, chunk)], sc_v)

        def group2(g, carry):
            _topk_group(sc_v, idx_v, w_v, 2 * g)
            _topk_group(sc_v, idx_v, w_v, 2 * g + 1)
            return carry

        lax.fori_loop(0, ngroups // 2, group2, 0)
        pltpu.sync_copy(idx_v, idx_hbm.at[pl.ds(base, chunk)])
        pltpu.sync_copy(w_v, w_hbm.at[pl.ds(base, chunk)])

    return sc_topk


_sc_topk_by_size = {n: _make_sc_topk(n) for n in set(_CHAIN)}


def _make_scores_call(block_off, ntok):
    nsteps = ntok // _BT

    return pl.pallas_call(
        _scores_body,
        grid=(nsteps,),
        in_specs=[
            pl.BlockSpec((_BT, HIDDEN), lambda i, o=block_off: (o + i, 0)),
            pl.BlockSpec((NUM_EXPERTS, HIDDEN), lambda i: (0, 0)),
        ],
        out_specs=pl.BlockSpec((_BT, NUM_EXPERTS), lambda i: (i, 0)),
        out_shape=jax.ShapeDtypeStruct((ntok, NUM_EXPERTS), jnp.float32),
    )


def kernel(hidden_states, weight):
    idxs, ws = [], []
    off = 0
    for ntok in _CHAIN:
        scores_c = _make_scores_call(off // _BT, ntok)(hidden_states, weight)
        i_c, w_c = _sc_topk_by_size[ntok](scores_c)
        idxs.append(i_c)
        ws.append(w_c)
        off += ntok
    return jnp.concatenate(idxs, 0), jnp.concatenate(ws, 0)


# speedup vs baseline: 1.2441x; 1.0846x over previous
import functools

import jax
import jax.numpy as jnp
from jax import lax
from jax.experimental import pallas as pl
from jax.experimental.pallas import tpu as pltpu
from jax.experimental.pallas import tpu_sc as plsc

NUM_EXPERTS = 64
K_TOP = 8
HIDDEN = 4096
TOKENS = 16384

_BT = 1024
_CHAIN = (5120, 5120, 5120, 1024)

_NC = 2
_NS = 16
_L = 16
_NW = _NC * _NS


def _scores_body(x_ref, w_ref, o_ref):
    logits = lax.dot_general(
        x_ref[...], w_ref[...],
        (((1,), (1,)), ((), ())),
        preferred_element_type=jnp.float32,
    )
    m = jnp.max(logits, axis=1, keepdims=True)
    p = jnp.exp(logits - m)
    o_ref[...] = (p / jnp.sum(p, axis=1, keepdims=True)).T


def _topk_group(sc_v, idx_v, w_v, g):
    span = pl.ds(g * _L, _L)
    keys = []
    for e in range(NUM_EXPERTS):
        b = plsc.bitcast(sc_v[e, span], jnp.uint32)
        keys.append((b & jnp.uint32(0xFFFFFFC0)) | jnp.uint32(63 - e))

    def ce(a, i, j):
        hi = jnp.maximum(a[i], a[j])
        lo = jnp.minimum(a[i], a[j])
        a[i], a[j] = hi, lo

    s8 = ((0, 1), (2, 3), (4, 5), (6, 7),
          (0, 2), (1, 3), (4, 6), (5, 7),
          (1, 2), (5, 6), (0, 4), (3, 7),
          (1, 5), (2, 6), (1, 4), (3, 6),
          (2, 4), (3, 5), (3, 4))
    blocks = []
    for blk in range(NUM_EXPERTS // 8):
        a = keys[8 * blk:8 * blk + 8]
        for i, j in s8:
            ce(a, i, j)
        blocks.append(a)
    bit12 = ((0, 4), (1, 5), (2, 6), (3, 7),
             (0, 2), (1, 3), (4, 6), (5, 7),
             (0, 1), (2, 3), (4, 5), (6, 7))
    while len(blocks) > 1:
        nxt = []
        for p in range(0, len(blocks), 2):
            a, b = blocks[p], blocks[p + 1]
            c = [jnp.maximum(a[i], b[7 - i]) for i in range(8)]
            for i, j in bit12:
                ce(c, i, j)
            nxt.append(c)
        blocks = nxt
    top = blocks[0]
    tokloc = lax.iota(jnp.int32, _L) + g * _L
    for j in range(K_TOP):
        ij = jnp.int32(63) - plsc.bitcast(top[j] & jnp.uint32(63), jnp.int32)
        wj = plsc.load_gather(sc_v, [ij, tokloc])
        idx_v[j, span] = ij
        w_v[j, span] = wj


def _make_sc_topk(ntok):
    chunk = ntok // _NW
    ngroups = chunk // _L

    @functools.partial(
        pl.kernel,
        out_type=(
            jax.ShapeDtypeStruct((K_TOP, ntok), jnp.int32),
            jax.ShapeDtypeStruct((K_TOP, ntok), jnp.float32),
        ),
        mesh=plsc.VectorSubcoreMesh(core_axis_name="c", subcore_axis_name="s"),
        compiler_params=pltpu.CompilerParams(
            needs_layout_passes=False, use_tc_tiling_on_sc=False),
        scratch_types=[
            pltpu.VMEM((NUM_EXPERTS, chunk), jnp.float32),
            pltpu.VMEM((K_TOP, chunk), jnp.int32),
            pltpu.VMEM((K_TOP, chunk), jnp.float32),
        ],
    )
    def sc_topk(scores_hbm, idx_hbm, w_hbm, sc_v, idx_v, w_v):
        wid = lax.axis_index("s") * _NC + lax.axis_index("c")
        base = wid * chunk
        pltpu.sync_copy(scores_hbm.at[:, pl.ds(base, chunk)], sc_v)

        def group2(g, carry):
            _topk_group(sc_v, idx_v, w_v, 2 * g)
            _topk_group(sc_v, idx_v, w_v, 2 * g + 1)
            return carry

        lax.fori_loop(0, ngroups // 2, group2, 0)
        pltpu.sync_copy(idx_v, idx_hbm.at[:, pl.ds(base, chunk)])
        pltpu.sync_copy(w_v, w_hbm.at[:, pl.ds(base, chunk)])

    return sc_topk


_sc_topk_by_size = {n: _make_sc_topk(n) for n in set(_CHAIN)}


def _make_scores_call(block_off, ntok):
    nsteps = ntok // _BT

    return pl.pallas_call(
        _scores_body,
        grid=(nsteps,),
        in_specs=[
            pl.BlockSpec((_BT, HIDDEN), lambda i, o=block_off: (o + i, 0)),
            pl.BlockSpec((NUM_EXPERTS, HIDDEN), lambda i: (0, 0)),
        ],
        out_specs=pl.BlockSpec((NUM_EXPERTS, _BT), lambda i: (0, i)),
        out_shape=jax.ShapeDtypeStruct((NUM_EXPERTS, ntok), jnp.float32),
    )


def kernel(hidden_states, weight):
    idxs, ws = [], []
    off = 0
    for ntok in _CHAIN:
        scores_c = _make_scores_call(off // _BT, ntok)(hidden_states, weight)
        i_c, w_c = _sc_topk_by_size[ntok](scores_c)
        idxs.append(i_c)
        ws.append(w_c)
        off += ntok
    idx_t = jnp.concatenate(idxs, axis=1)
    w_t = jnp.concatenate(ws, axis=1)
    return idx_t.T, w_t.T
